# baseline (speedup 1.0000x reference)
import jax
import jax.numpy as jnp
from jax import lax
from jax.experimental import pallas as pl
from jax.experimental.pallas import tpu as pltpu
from jax.experimental.pallas import tpu_sc as plsc

N = 10000
CUT = 32
D = 128
DH = D // 2
NC = 2
NS = 16
NW = NC * NS

C_NODES = 4
C_EDGES = C_NODES * CUT
NUM_CHUNKS = N // C_NODES
BASE_CHUNKS = NUM_CHUNKS // NW
EXTRA = NUM_CHUNKS - BASE_CHUNKS * NW
MAX_CHUNKS = BASE_CHUNKS + 1
NBUF = 4
PREF = 2
NGROUPS = -(-MAX_CHUNKS // NBUF)
PAD_EDGES = MAX_CHUNKS * NW * C_EDGES


def _sc_body(feat_hbm, swz_hbm, gflat_hbm, out_hbm, idx_all, rows_bf, rows_f,
             cent_v, gsem, csem, wsem):
    wid = lax.axis_index("s") * NC + lax.axis_index("c")
    s0 = BASE_CHUNKS * wid + jnp.minimum(wid, EXTRA)
    cw = BASE_CHUNKS + jnp.where(wid < EXTRA, 1, 0)

    pltpu.sync_copy(gflat_hbm.at[pl.ds(s0 * C_EDGES, MAX_CHUNKS * C_EDGES)],
                    idx_all)

    def start_fetch(t, b):
        pltpu.make_async_copy(
            swz_hbm.at[idx_all.at[pl.ds(t * C_EDGES, C_EDGES)]],
            rows_bf.at[b], gsem).start()
        pltpu.make_async_copy(
            feat_hbm.at[pl.ds((s0 + t) * C_NODES, C_NODES)],
            cent_v.at[b], csem).start()

    for tt in range(PREF):
        start_fetch(jnp.int32(tt), tt)

    def group_body(g, carry):
        for b in range(NBUF):
            t = g * NBUF + b

            @pl.when(t < cw)
            def _(t=t, b=b):
                pltpu.make_async_copy(
                    feat_hbm.at[pl.ds(s0 * C_NODES, C_NODES)],
                    cent_v.at[b], csem).wait()
                pltpu.make_async_copy(
                    swz_hbm.at[idx_all.at[pl.ds(0, C_EDGES)]],
                    rows_bf.at[b], gsem).wait()

                shv = jnp.full((16,), 16, dtype=jnp.int32)
                msk = jnp.full((16,), -65536, dtype=jnp.int32)
                for i in range(C_NODES):
                    clo = [cent_v[b, i, pl.ds(dv * 16, 16)]
                           for dv in range(4)]
                    chi = [cent_v[b, i, pl.ds(DH + dv * 16, 16)]
                           for dv in range(4)]

                    def edge_body(c, _, b=b, i=i, clo=clo, chi=chi,
                                  shv=shv, msk=msk):
                        r = i * CUT + c
                        for dv in range(4):
                            pi = rows_bf[b, r, pl.ds(dv * 16, 16)]
                            ev = lax.bitcast_convert_type(
                                lax.shift_left(pi, shv), jnp.float32)
                            od = lax.bitcast_convert_type(
                                lax.bitwise_and(pi, msk), jnp.float32)
                            rows_f[b, r, pl.ds(dv * 16, 16)] = ev - clo[dv]
                            rows_f[b, r, pl.ds(DH + dv * 16, 16)] = (
                                od - chi[dv])
                        return 0

                    lax.fori_loop(0, CUT, edge_body, 0)

                pltpu.make_async_copy(
                    rows_f.at[b],
                    out_hbm.at[pl.ds((s0 + t) * C_EDGES, C_EDGES)],
                    wsem).start()

                @pl.when(t + PREF < cw)
                def _(t=t, b=b):
                    @pl.when(t >= NBUF - PREF)
                    def _():
                        pltpu.make_async_copy(
                            rows_f.at[b],
                            out_hbm.at[pl.ds(s0 * C_EDGES, C_EDGES)],
                            wsem).wait()
                    start_fetch(t + PREF, (b + PREF) % NBUF)

        return carry

    lax.fori_loop(0, NGROUPS, group_body, 0)

    for _ in range(NBUF):
        pltpu.make_async_copy(
            rows_f.at[0],
            out_hbm.at[pl.ds(s0 * C_EDGES, C_EDGES)], wsem).wait()


@jax.jit
def _sc_expand(feat, swz, gflat_padded):
    mesh = plsc.VectorSubcoreMesh(core_axis_name="c", subcore_axis_name="s")
    return pl.kernel(
        _sc_body,
        mesh=mesh,
        compiler_params=pltpu.CompilerParams(use_tc_tiling_on_sc=False),
        out_type=jax.ShapeDtypeStruct((N * CUT, D), jnp.float32),
        scratch_types=[
            pltpu.VMEM((MAX_CHUNKS * C_EDGES,), jnp.int32),
            pltpu.VMEM((NBUF, C_EDGES, DH), jnp.int32),
            pltpu.VMEM((NBUF, C_EDGES, D), jnp.float32),
            pltpu.VMEM((NBUF, C_NODES, D), jnp.float32),
            pltpu.SemaphoreType.DMA,
            pltpu.SemaphoreType.DMA,
            pltpu.SemaphoreType.DMA,
        ],
    )(feat, swz, gflat_padded)


def kernel(x_features, x_graph):
    feat = x_features.reshape(N, D)
    fb = feat.astype(jnp.bfloat16)
    swz = lax.bitcast_convert_type(
        jnp.stack([fb[:, :DH], fb[:, DH:]], axis=-1), jnp.int32)
    gflat = x_graph.astype(jnp.int32).reshape(N * CUT)
    gflat = jnp.pad(gflat, (0, PAD_EDGES - N * CUT))
    out = _sc_expand(feat, swz, gflat)
    return out.reshape(1, N, CUT, D)


# speedup vs baseline: 1.8558x; 1.8558x over previous
import jax
import jax.numpy as jnp
from jax import lax
from jax.experimental import pallas as pl
from jax.experimental.pallas import tpu as pltpu
from jax.experimental.pallas import tpu_sc as plsc

N = 10000
CUT = 32
D = 128
NC = 2
NS = 16
NW = NC * NS

C_NODES = 8
C_EDGES = C_NODES * CUT
IDXW = 128
NUM_CHUNKS = N // C_NODES
BASE_CHUNKS = NUM_CHUNKS // NW
EXTRA = NUM_CHUNKS - BASE_CHUNKS * NW
MAX_CHUNKS = BASE_CHUNKS + 1
NBUF = 3
PREF = 2
NGROUPS = -(-MAX_CHUNKS // NBUF)
PAD_EDGES = MAX_CHUNKS * NW * C_EDGES


def _sc_body(feat_hbm, gflat_hbm, out_hbm, idx_all, rows_v, cent_v,
             gsem, csem, wsem):
    wid = lax.axis_index("s") * NC + lax.axis_index("c")
    s0 = BASE_CHUNKS * wid + jnp.minimum(wid, EXTRA)
    cw = BASE_CHUNKS + jnp.where(wid < EXTRA, 1, 0)

    pltpu.sync_copy(gflat_hbm.at[pl.ds(s0 * C_EDGES, MAX_CHUNKS * C_EDGES)],
                    idx_all)

    def start_fetch(t, b):
        for h in range(C_EDGES // IDXW):
            pltpu.make_async_copy(
                feat_hbm.at[idx_all.at[pl.ds(t * C_EDGES + h * IDXW, IDXW)]],
                rows_v.at[b, pl.ds(h * IDXW, IDXW)], gsem).start()
        pltpu.make_async_copy(
            feat_hbm.at[pl.ds((s0 + t) * C_NODES, C_NODES)],
            cent_v.at[b], csem).start()

    for tt in range(PREF):
        start_fetch(jnp.int32(tt), tt)

    def group_body(g, carry):
        for b in range(NBUF):
            t = g * NBUF + b

            @pl.when(t < cw)
            def _(t=t, b=b):
                pltpu.make_async_copy(
                    feat_hbm.at[pl.ds(s0 * C_NODES, C_NODES)],
                    cent_v.at[b], csem).wait()
                for h in range(C_EDGES // IDXW):
                    pltpu.make_async_copy(
                        feat_hbm.at[idx_all.at[pl.ds(0, IDXW)]],
                        rows_v.at[b, pl.ds(0, IDXW)], gsem).wait()

                for i in range(C_NODES):
                    cvecs = [cent_v[b, i, pl.ds(dv * 16, 16)]
                             for dv in range(8)]

                    def edge_body(c, _, b=b, i=i, cvecs=cvecs):
                        r = i * CUT + c
                        for dv in range(8):
                            sl = pl.ds(dv * 16, 16)
                            rows_v[b, r, sl] = rows_v[b, r, sl] - cvecs[dv]
                        return 0

                    lax.fori_loop(0, CUT, edge_body, 0)

                pltpu.make_async_copy(
                    rows_v.at[b],
                    out_hbm.at[pl.ds((s0 + t) * C_EDGES, C_EDGES)],
                    wsem).start()

                @pl.when(t + PREF < cw)
                def _(t=t, b=b):
                    @pl.when(t >= NBUF - PREF)
                    def _():
                        pltpu.make_async_copy(
                            rows_v.at[b],
                            out_hbm.at[pl.ds(s0 * C_EDGES, C_EDGES)],
                            wsem).wait()
                    start_fetch(t + PREF, (b + PREF) % NBUF)

        return carry

    lax.fori_loop(0, NGROUPS, group_body, 0)

    for _ in range(NBUF):
        pltpu.make_async_copy(
            rows_v.at[0],
            out_hbm.at[pl.ds(s0 * C_EDGES, C_EDGES)], wsem).wait()


@jax.jit
def _sc_expand(feat, gflat_padded):
    mesh = plsc.VectorSubcoreMesh(core_axis_name="c", subcore_axis_name="s")
    return pl.kernel(
        _sc_body,
        mesh=mesh,
        out_type=jax.ShapeDtypeStruct((N * CUT, D), jnp.float32),
        scratch_types=[
            pltpu.VMEM((MAX_CHUNKS * C_EDGES,), jnp.int32),
            pltpu.VMEM((NBUF, C_EDGES, D), jnp.float32),
            pltpu.VMEM((NBUF, C_NODES, D), jnp.float32),
            pltpu.SemaphoreType.DMA,
            pltpu.SemaphoreType.DMA,
            pltpu.SemaphoreType.DMA,
        ],
    )(feat, gflat_padded)


def kernel(x_features, x_graph):
    feat = x_features.reshape(N, D)
    gflat = x_graph.astype(jnp.int32).reshape(N * CUT)
    gflat = jnp.pad(gflat, (0, PAD_EDGES - N * CUT))
    out = _sc_expand(feat, gflat)
    return out.reshape(1, N, CUT, D)


# speedup vs baseline: 1.8754x; 1.0106x over previous
import jax
import jax.numpy as jnp
from jax import lax
from jax.experimental import pallas as pl
from jax.experimental.pallas import tpu as pltpu
from jax.experimental.pallas import tpu_sc as plsc

N = 10000
CUT = 32
D = 128
NC = 2
NS = 16
NW = NC * NS

C_NODES = 4
C_EDGES = C_NODES * CUT
NUM_CHUNKS = N // C_NODES
BASE_CHUNKS = NUM_CHUNKS // NW
EXTRA = NUM_CHUNKS - BASE_CHUNKS * NW
MAX_CHUNKS = BASE_CHUNKS + 1
NBUF = 6
PREF = 3
NGROUPS = -(-MAX_CHUNKS // NBUF)
PAD_EDGES = MAX_CHUNKS * NW * C_EDGES


def _sc_body(feat_hbm, gflat_hbm, out_hbm, idx_all, rows_v, cent_v,
             gsem, csem, wsem):
    wid = lax.axis_index("s") * NC + lax.axis_index("c")
    s0 = BASE_CHUNKS * wid + jnp.minimum(wid, EXTRA)
    cw = BASE_CHUNKS + jnp.where(wid < EXTRA, 1, 0)

    pltpu.sync_copy(gflat_hbm.at[pl.ds(s0 * C_EDGES, MAX_CHUNKS * C_EDGES)],
                    idx_all)

    def start_fetch(t, b):
        pltpu.make_async_copy(
            feat_hbm.at[idx_all.at[pl.ds(t * C_EDGES, C_EDGES)]],
            rows_v.at[b], gsem).start()
        pltpu.make_async_copy(
            feat_hbm.at[pl.ds((s0 + t) * C_NODES, C_NODES)],
            cent_v.at[b], csem).start()

    for tt in range(PREF):
        start_fetch(jnp.int32(tt), tt)

    def group_body(g, carry):
        for b in range(NBUF):
            t = g * NBUF + b

            @pl.when(t < cw)
            def _(t=t, b=b):
                pltpu.make_async_copy(
                    feat_hbm.at[pl.ds(s0 * C_NODES, C_NODES)],
                    cent_v.at[b], csem).wait()
                pltpu.make_async_copy(
                    feat_hbm.at[idx_all.at[pl.ds(0, C_EDGES)]],
                    rows_v.at[b], gsem).wait()

                for i in range(C_NODES):
                    cvecs = [cent_v[b, i, pl.ds(dv * 16, 16)]
                             for dv in range(8)]

                    def edge_body(c, _, b=b, i=i, cvecs=cvecs):
                        r = i * CUT + c
                        for dv in range(8):
                            sl = pl.ds(dv * 16, 16)
                            rows_v[b, r, sl] = rows_v[b, r, sl] - cvecs[dv]
                        return 0

                    lax.fori_loop(0, CUT, edge_body, 0)

                pltpu.make_async_copy(
                    rows_v.at[b],
                    out_hbm.at[pl.ds((s0 + t) * C_EDGES, C_EDGES)],
                    wsem).start()

                @pl.when(t + PREF < cw)
                def _(t=t, b=b):
                    @pl.when(t >= NBUF - PREF)
                    def _():
                        pltpu.make_async_copy(
                            rows_v.at[b],
                            out_hbm.at[pl.ds(s0 * C_EDGES, C_EDGES)],
                            wsem).wait()
                    start_fetch(t + PREF, (b + PREF) % NBUF)

        return carry

    lax.fori_loop(0, NGROUPS, group_body, 0)

    for _ in range(NBUF):
        pltpu.make_async_copy(
            rows_v.at[0],
            out_hbm.at[pl.ds(s0 * C_EDGES, C_EDGES)], wsem).wait()


@jax.jit
def _sc_expand(feat, gflat_padded):
    mesh = plsc.VectorSubcoreMesh(core_axis_name="c", subcore_axis_name="s")
    return pl.kernel(
        _sc_body,
        mesh=mesh,
        out_type=jax.ShapeDtypeStruct((N * CUT, D), jnp.float32),
        scratch_types=[
            pltpu.VMEM((MAX_CHUNKS * C_EDGES,), jnp.int32),
            pltpu.VMEM((NBUF, C_EDGES, D), jnp.float32),
            pltpu.VMEM((NBUF, C_NODES, D), jnp.float32),
            pltpu.SemaphoreType.DMA,
            pltpu.SemaphoreType.DMA,
            pltpu.SemaphoreType.DMA,
        ],
    )(feat, gflat_padded)


def kernel(x_features, x_graph):
    feat = x_features.reshape(N, D)
    gflat = x_graph.astype(jnp.int32).reshape(N * CUT)
    gflat = jnp.pad(gflat, (0, PAD_EDGES - N * CUT))
    out = _sc_expand(feat, gflat)
    return out.reshape(1, N, CUT, D)
